# baseline (speedup 1.0000x reference)
import functools
import jax
import jax.numpy as jnp
from jax import lax
from jax.experimental import pallas as pl
from jax.experimental.pallas import tpu as pltpu
from jax.experimental.pallas import tpu_sc as plsc

N = 10000
D = 128
OUT = 128
E = 320000

NS = 16
CH = 128
G = 16
CPT = 157
NG = (CPT + G - 1) // G
LASTG = CPT - (NG - 1) * G
E_PAD = NS * CH * CPT
PAD_E = E_PAD - E
N_PAD = 10112
ROWS_PT = N_PAD // NS

BR = 1000


def _counts_body(s1, r1, s2, r2, zeros_hbm, ones_hbm, c0, c1, c2, c3,
                 acc0, acc1, ones, idxv, zvm, sem):
    c = lax.axis_index("c")
    s = lax.axis_index("s")
    pltpu.sync_copy(ones_hbm, ones)
    row = pl.ds(s * ROWS_PT, ROWS_PT)
    pltpu.sync_copy(zeros_hbm, zvm)
    pltpu.sync_copy(zvm, acc0.at[row])
    pltpu.sync_copy(zvm, acc1.at[row])
    plsc.subcore_barrier()

    def count_into(idx_hbm, acc, sem):
        pltpu.sync_copy(idx_hbm.at[s], idxv)

        for g in range(NG):
            pend = [
                pltpu.async_copy(ones, acc.at[idxv.at[g * G + j]], sem,
                                 add=True)
                for j in range(G if g < NG - 1 else LASTG)
            ]
            for p in pend:
                p.wait()

    def copyout(acc, out):
        pltpu.sync_copy(acc.at[row], zvm)
        pltpu.sync_copy(zvm, out.at[row])

    @pl.when(c == 0)
    def _():
        count_into(s1, acc0, sem)
        count_into(r1, acc1, sem)
        plsc.subcore_barrier()
        copyout(acc0, c0)
        copyout(acc1, c1)

    @pl.when(c == 1)
    def _():
        count_into(s2, acc0, sem)
        count_into(r2, acc1, sem)
        plsc.subcore_barrier()
        copyout(acc0, c2)
        copyout(acc1, c3)


def _sc_counts(s1, r1, s2, r2, zeros1, ones2):
    mesh = plsc.VectorSubcoreMesh(core_axis_name="c", subcore_axis_name="s")
    f = pl.kernel(
        _counts_body,
        out_type=[jax.ShapeDtypeStruct((N_PAD, 1), jnp.float32)] * 4,
        mesh=mesh,
        scratch_types=[
            pltpu.VMEM_SHARED((N_PAD, 1), jnp.float32),
            pltpu.VMEM_SHARED((N_PAD, 1), jnp.float32),
            pltpu.VMEM((CH, 1), jnp.float32),
            pltpu.VMEM((CPT, CH), jnp.int32),
            pltpu.VMEM((ROWS_PT, 1), jnp.float32),
            pltpu.SemaphoreType.DMA,
        ],
    )
    return f(s1, r1, s2, r2, zeros1, ones2)


def _agg_body(h1, h2, s1, r1, s2, r2, zeros2, out1, out2,
              acc, sbuf0, sbuf1, rbuf0, rbuf1, rows0, rows1,
              gsem0, gsem1, ssem0, ssem1, isem0, isem1):
    c = lax.axis_index("c")
    s = lax.axis_index("s")
    row = pl.ds(s * ROWS_PT, ROWS_PT)
    pltpu.sync_copy(zeros2, acc.at[row])
    rows = (rows0, rows1)
    gsem = (gsem0, gsem1)
    ssem = (ssem0, ssem1)
    sbuf = (sbuf0, sbuf1)
    rbuf = (rbuf0, rbuf1)

    def run(h, sidx, ridx, out):
        pltpu.sync_copy(sidx.at[s, pl.ds(0, G)], sbuf[0])
        pltpu.sync_copy(ridx.at[s, pl.ds(0, G)], rbuf[0])
        plsc.subcore_barrier()
        pend_g = [None, None]
        pend_s = [None, None]
        pend_i = [None, None]
        pend_g[0] = pltpu.async_copy(h.at[sbuf[0].at[0]], rows[0], gsem[0])
        for jj in range(CPT):
            g, j = divmod(jj, G)
            ib = sbuf[g % 2], rbuf[g % 2]
            if j == 0 and g + 1 < NG:
                gl = G if g + 1 < NG - 1 else LASTG
                pend_i[0] = pltpu.async_copy(
                    sidx.at[s, pl.ds((g + 1) * G, gl)],
                    sbuf[(g + 1) % 2].at[pl.ds(0, gl)], isem0)
                pend_i[1] = pltpu.async_copy(
                    ridx.at[s, pl.ds((g + 1) * G, gl)],
                    rbuf[(g + 1) % 2].at[pl.ds(0, gl)], isem1)
            if jj >= 1:
                pend_s[(jj - 1) % 2].wait()
            if jj + 1 < CPT:
                ng, nj = divmod(jj + 1, G)
                if nj == 0:
                    pend_i[0].wait()
                    pend_i[1].wait()
                pend_g[(jj + 1) % 2] = pltpu.async_copy(
                    h.at[sbuf[ng % 2].at[nj]], rows[(jj + 1) % 2],
                    gsem[(jj + 1) % 2])
            pend_g[jj % 2].wait()
            pend_s[jj % 2] = pltpu.async_copy(
                rows[jj % 2], acc.at[ib[1].at[j]], ssem[jj % 2], add=True)
        pend_s[(CPT - 1) % 2].wait()
        plsc.subcore_barrier()
        pltpu.sync_copy(acc.at[row], out.at[row])

    @pl.when(c == 0)
    def _():
        run(h1, s1, r1, out1)

    @pl.when(c == 1)
    def _():
        run(h2, s2, r2, out2)


def _sc_aggregate(h1s, h2s, s1, r1, s2, r2, zeros2):
    mesh = plsc.VectorSubcoreMesh(core_axis_name="c", subcore_axis_name="s")
    f = pl.kernel(
        _agg_body,
        out_type=[jax.ShapeDtypeStruct((N_PAD, D), jnp.float32)] * 2,
        mesh=mesh,
        scratch_types=[
            pltpu.VMEM_SHARED((N_PAD, D), jnp.float32),
            pltpu.VMEM((G, CH), jnp.int32),
            pltpu.VMEM((G, CH), jnp.int32),
            pltpu.VMEM((G, CH), jnp.int32),
            pltpu.VMEM((G, CH), jnp.int32),
            pltpu.VMEM((CH, D), jnp.float32),
            pltpu.VMEM((CH, D), jnp.float32),
            pltpu.SemaphoreType.DMA,
            pltpu.SemaphoreType.DMA,
            pltpu.SemaphoreType.DMA,
            pltpu.SemaphoreType.DMA,
            pltpu.SemaphoreType.DMA,
            pltpu.SemaphoreType.DMA,
        ],
    )
    return f(h1s, h2s, s1, r1, s2, r2, zeros2)


def _mm_body(nodes, w1, b1, w2, b2, o1, o2):
    x = nodes[...]
    o1[...] = jnp.dot(x, w1[...], preferred_element_type=jnp.float32) + b1[...]
    o2[...] = jnp.dot(x, w2[...], preferred_element_type=jnp.float32) + b2[...]


def _tc_mm(nodes, W1, b1, W2, b2):
    grid = (N // BR,)
    rb = pl.BlockSpec((BR, D), lambda i: (i, 0))
    full = pl.BlockSpec((D, OUT), lambda i: (0, 0))
    bias = pl.BlockSpec((1, OUT), lambda i: (0, 0))
    ob = pl.BlockSpec((BR, OUT), lambda i: (i, 0))
    return pl.pallas_call(
        _mm_body,
        grid=grid,
        in_specs=[rb, full, bias, full, bias],
        out_specs=[ob, ob],
        out_shape=[jax.ShapeDtypeStruct((N, OUT), jnp.float32)] * 2,
    )(nodes, W1, b1.reshape(1, OUT), W2, b2.reshape(1, OUT))


def _scale_body(h1u, h2u, cnt, o1, o2):
    i = pl.program_id(0)
    gidx = i * BR + lax.broadcasted_iota(jnp.int32, (BR, 1), 0)
    corr = jnp.where(gidx < PAD_E, 1.0, 0.0).astype(jnp.float32)
    o1[...] = h1u[...] * lax.rsqrt(cnt[:, 0:1] - corr + 1.0)
    o2[...] = h2u[...] * lax.rsqrt(cnt[:, 2:3] - corr + 1.0)


def _tc_scale(h1u, h2u, cnt):
    grid = (N // BR,)
    hb = pl.BlockSpec((BR, OUT), lambda i: (i, 0))
    cb = pl.BlockSpec((BR, 4), lambda i: (i, 0))
    return pl.pallas_call(
        _scale_body,
        grid=grid,
        in_specs=[hb, hb, cb],
        out_specs=[hb, hb],
        out_shape=[jax.ShapeDtypeStruct((N, OUT), jnp.float32)] * 2,
    )(h1u, h2u, cnt)


def _post_body(agg1, agg2, h1s, h2s, cnt, w3, b3, out):
    r1 = lax.rsqrt(cnt[:, 1:2] + 1.0)
    r2 = lax.rsqrt(cnt[:, 3:4] + 1.0)
    a1 = (agg1[...] + h1s[...]) * r1
    a2 = (agg2[...] + h2s[...]) * r2
    y = jnp.dot(a1, w3[0:OUT, :], preferred_element_type=jnp.float32)
    y = y + jnp.dot(a2, w3[OUT:2 * OUT, :], preferred_element_type=jnp.float32)
    out[...] = jnp.maximum(y + b3[...], 0.0)


def _tc_post(agg1, agg2, h1s, h2s, cnt, W3, b3):
    grid = (N // BR,)
    ab = pl.BlockSpec((BR, D), lambda i: (i, 0))
    cb = pl.BlockSpec((BR, 4), lambda i: (i, 0))
    wb = pl.BlockSpec((2 * OUT, OUT), lambda i: (0, 0))
    bias = pl.BlockSpec((1, OUT), lambda i: (0, 0))
    ob = pl.BlockSpec((BR, OUT), lambda i: (i, 0))
    return pl.pallas_call(
        _post_body,
        grid=grid,
        in_specs=[ab, ab, ab, ab, cb, wb, bias],
        out_specs=ob,
        out_shape=jax.ShapeDtypeStruct((N, OUT), jnp.float32),
    )(agg1, agg2, h1s, h2s, cnt, W3, b3.reshape(1, OUT))


def _pad_idx(idx, base, mod):
    p = base + jnp.arange(PAD_E, dtype=jnp.int32) % mod
    return jnp.concatenate([idx.astype(jnp.int32), p]).reshape(NS, CPT, CH)


@jax.jit
def kernel(nodes, senders, receivers, grid_senders, grid_receivers,
           W1, b1, W2, b2, W3, b3):
    s1 = _pad_idx(senders, 0, N)
    r1 = _pad_idx(receivers, N, N_PAD - N)
    s2 = _pad_idx(grid_senders, 0, N)
    r2 = _pad_idx(grid_receivers, N, N_PAD - N)

    h1u, h2u = _tc_mm(nodes, W1, b1, W2, b2)

    zeros1 = jnp.zeros((ROWS_PT, 1), jnp.float32)
    ones2 = jnp.ones((CH, 1), jnp.float32)
    c0, c1, c2, c3 = _sc_counts(s1, r1, s2, r2, zeros1, ones2)
    cnt = jnp.concatenate([c0, c1, c2, c3], axis=1)

    h1s, h2s = _tc_scale(h1u, h2u, cnt)

    zeros2 = jnp.zeros((ROWS_PT, D), jnp.float32)
    agg1, agg2 = _sc_aggregate(h1s, h2s, s1, r1, s2, r2, zeros2)

    return _tc_post(agg1, agg2, h1s, h2s, cnt, W3, b3)


# speedup vs baseline: 1.1459x; 1.1459x over previous
import functools
import jax
import jax.numpy as jnp
from jax import lax
from jax.experimental import pallas as pl
from jax.experimental.pallas import tpu as pltpu
from jax.experimental.pallas import tpu_sc as plsc

N = 10000
D = 128
OUT = 128
E = 320000

NS = 16
CH = 128
G = 16
CPT = 157
NG = (CPT + G - 1) // G
LASTG = CPT - (NG - 1) * G
E_PAD = NS * CH * CPT
PAD_E = E_PAD - E
N_PAD = 10112
ROWS_PT = N_PAD // NS

BR = 1000


def _counts_body(s1, r1, s2, r2, zeros_hbm, c0, c1, c2, c3,
                 acc0, acc1, ones, idxv, zvm, sem):
    c = lax.axis_index("c")
    s = lax.axis_index("s")
    for k in range(CH // 16):
        ones[pl.ds(k * 16, 16)] = jnp.ones((16,), jnp.float32)
    row = pl.ds(s * ROWS_PT, ROWS_PT)
    pltpu.sync_copy(zeros_hbm, zvm)
    pltpu.sync_copy(zvm, acc0.at[row])
    pltpu.sync_copy(zvm, acc1.at[row])
    plsc.subcore_barrier()

    def count_into(idx_hbm, acc, sem):
        pltpu.sync_copy(idx_hbm.at[s], idxv)

        for g in range(NG):
            pend = [
                pltpu.async_copy(ones, acc.at[idxv.at[g * G + j]], sem,
                                 add=True)
                for j in range(G if g < NG - 1 else LASTG)
            ]
            for p in pend:
                p.wait()

    def copyout(acc, out):
        pltpu.sync_copy(acc.at[row], zvm)
        pltpu.sync_copy(zvm, out.at[row])

    @pl.when(c == 0)
    def _():
        count_into(s1, acc0, sem)
        count_into(r1, acc1, sem)
        plsc.subcore_barrier()
        copyout(acc0, c0)
        copyout(acc1, c1)

    @pl.when(c == 1)
    def _():
        count_into(s2, acc0, sem)
        count_into(r2, acc1, sem)
        plsc.subcore_barrier()
        copyout(acc0, c2)
        copyout(acc1, c3)


def _sc_counts(s1, r1, s2, r2, zeros1):
    mesh = plsc.VectorSubcoreMesh(core_axis_name="c", subcore_axis_name="s")
    f = pl.kernel(
        _counts_body,
        out_type=[jax.ShapeDtypeStruct((N_PAD,), jnp.float32)] * 4,
        mesh=mesh,
        scratch_types=[
            pltpu.VMEM_SHARED((N_PAD,), jnp.float32),
            pltpu.VMEM_SHARED((N_PAD,), jnp.float32),
            pltpu.VMEM((CH,), jnp.float32),
            pltpu.VMEM((CPT, CH), jnp.int32),
            pltpu.VMEM((ROWS_PT,), jnp.float32),
            pltpu.SemaphoreType.DMA,
        ],
    )
    return f(s1, r1, s2, r2, zeros1)


def _agg_body(h1, h2, s1, r1, s2, r2, zeros2, out1, out2,
              acc, sbuf0, sbuf1, rbuf0, rbuf1, rows0, rows1,
              gsem0, gsem1, ssem0, ssem1, isem0, isem1):
    c = lax.axis_index("c")
    s = lax.axis_index("s")
    row = pl.ds(s * ROWS_PT, ROWS_PT)
    pltpu.sync_copy(zeros2, acc.at[row])
    rows = (rows0, rows1)
    gsem = (gsem0, gsem1)
    ssem = (ssem0, ssem1)
    sbuf = (sbuf0, sbuf1)
    rbuf = (rbuf0, rbuf1)

    def run(h, sidx, ridx, out):
        pltpu.sync_copy(sidx.at[s, pl.ds(0, G)], sbuf[0])
        pltpu.sync_copy(ridx.at[s, pl.ds(0, G)], rbuf[0])
        plsc.subcore_barrier()
        pend_g = [None, None]
        pend_s = [None, None]
        pend_i = [None, None]
        pend_g[0] = pltpu.async_copy(h.at[sbuf[0].at[0]], rows[0], gsem[0])
        for jj in range(CPT):
            g, j = divmod(jj, G)
            ib = sbuf[g % 2], rbuf[g % 2]
            if j == 0 and g + 1 < NG:
                gl = G if g + 1 < NG - 1 else LASTG
                pend_i[0] = pltpu.async_copy(
                    sidx.at[s, pl.ds((g + 1) * G, gl)],
                    sbuf[(g + 1) % 2].at[pl.ds(0, gl)], isem0)
                pend_i[1] = pltpu.async_copy(
                    ridx.at[s, pl.ds((g + 1) * G, gl)],
                    rbuf[(g + 1) % 2].at[pl.ds(0, gl)], isem1)
            if jj >= 1:
                pend_s[(jj - 1) % 2].wait()
            if jj + 1 < CPT:
                ng, nj = divmod(jj + 1, G)
                if nj == 0:
                    pend_i[0].wait()
                    pend_i[1].wait()
                pend_g[(jj + 1) % 2] = pltpu.async_copy(
                    h.at[sbuf[ng % 2].at[nj]], rows[(jj + 1) % 2],
                    gsem[(jj + 1) % 2])
            pend_g[jj % 2].wait()
            pend_s[jj % 2] = pltpu.async_copy(
                rows[jj % 2], acc.at[ib[1].at[j]], ssem[jj % 2], add=True)
        pend_s[(CPT - 1) % 2].wait()
        plsc.subcore_barrier()
        pltpu.sync_copy(acc.at[row], out.at[row])

    @pl.when(c == 0)
    def _():
        run(h1, s1, r1, out1)

    @pl.when(c == 1)
    def _():
        run(h2, s2, r2, out2)


def _sc_aggregate(h1s, h2s, s1, r1, s2, r2, zeros2):
    mesh = plsc.VectorSubcoreMesh(core_axis_name="c", subcore_axis_name="s")
    f = pl.kernel(
        _agg_body,
        out_type=[jax.ShapeDtypeStruct((N_PAD, D), jnp.float32)] * 2,
        mesh=mesh,
        scratch_types=[
            pltpu.VMEM_SHARED((N_PAD, D), jnp.float32),
            pltpu.VMEM((G, CH), jnp.int32),
            pltpu.VMEM((G, CH), jnp.int32),
            pltpu.VMEM((G, CH), jnp.int32),
            pltpu.VMEM((G, CH), jnp.int32),
            pltpu.VMEM((CH, D), jnp.float32),
            pltpu.VMEM((CH, D), jnp.float32),
            pltpu.SemaphoreType.DMA,
            pltpu.SemaphoreType.DMA,
            pltpu.SemaphoreType.DMA,
            pltpu.SemaphoreType.DMA,
            pltpu.SemaphoreType.DMA,
            pltpu.SemaphoreType.DMA,
        ],
    )
    return f(h1s, h2s, s1, r1, s2, r2, zeros2)


def _mm_body(nodes, w1, b1, w2, b2, o1, o2):
    x = nodes[...]
    o1[...] = jnp.dot(x, w1[...], preferred_element_type=jnp.float32) + b1[...]
    o2[...] = jnp.dot(x, w2[...], preferred_element_type=jnp.float32) + b2[...]


def _tc_mm(nodes, W1, b1, W2, b2):
    grid = (N // BR,)
    rb = pl.BlockSpec((BR, D), lambda i: (i, 0))
    full = pl.BlockSpec((D, OUT), lambda i: (0, 0))
    bias = pl.BlockSpec((1, OUT), lambda i: (0, 0))
    ob = pl.BlockSpec((BR, OUT), lambda i: (i, 0))
    return pl.pallas_call(
        _mm_body,
        grid=grid,
        in_specs=[rb, full, bias, full, bias],
        out_specs=[ob, ob],
        out_shape=[jax.ShapeDtypeStruct((N, OUT), jnp.float32)] * 2,
    )(nodes, W1, b1.reshape(1, OUT), W2, b2.reshape(1, OUT))


def _scale_body(h1u, h2u, cnt, o1, o2):
    i = pl.program_id(0)
    gidx = i * BR + lax.broadcasted_iota(jnp.int32, (BR, 1), 0)
    corr = jnp.where(gidx < PAD_E, 1.0, 0.0).astype(jnp.float32)
    o1[...] = h1u[...] * lax.rsqrt(cnt[:, 0:1] - corr + 1.0)
    o2[...] = h2u[...] * lax.rsqrt(cnt[:, 2:3] - corr + 1.0)


def _tc_scale(h1u, h2u, cnt):
    grid = (N // BR,)
    hb = pl.BlockSpec((BR, OUT), lambda i: (i, 0))
    cb = pl.BlockSpec((BR, 4), lambda i: (i, 0))
    return pl.pallas_call(
        _scale_body,
        grid=grid,
        in_specs=[hb, hb, cb],
        out_specs=[hb, hb],
        out_shape=[jax.ShapeDtypeStruct((N, OUT), jnp.float32)] * 2,
    )(h1u, h2u, cnt)


def _post_body(agg1, agg2, h1s, h2s, cnt, w3, b3, out):
    r1 = lax.rsqrt(cnt[:, 1:2] + 1.0)
    r2 = lax.rsqrt(cnt[:, 3:4] + 1.0)
    a1 = (agg1[...] + h1s[...]) * r1
    a2 = (agg2[...] + h2s[...]) * r2
    y = jnp.dot(a1, w3[0:OUT, :], preferred_element_type=jnp.float32)
    y = y + jnp.dot(a2, w3[OUT:2 * OUT, :], preferred_element_type=jnp.float32)
    out[...] = jnp.maximum(y + b3[...], 0.0)


def _tc_post(agg1, agg2, h1s, h2s, cnt, W3, b3):
    grid = (N // BR,)
    ab = pl.BlockSpec((BR, D), lambda i: (i, 0))
    cb = pl.BlockSpec((BR, 4), lambda i: (i, 0))
    wb = pl.BlockSpec((2 * OUT, OUT), lambda i: (0, 0))
    bias = pl.BlockSpec((1, OUT), lambda i: (0, 0))
    ob = pl.BlockSpec((BR, OUT), lambda i: (i, 0))
    return pl.pallas_call(
        _post_body,
        grid=grid,
        in_specs=[ab, ab, ab, ab, cb, wb, bias],
        out_specs=ob,
        out_shape=jax.ShapeDtypeStruct((N, OUT), jnp.float32),
    )(agg1, agg2, h1s, h2s, cnt, W3, b3.reshape(1, OUT))


def _pad_idx(idx, base, mod):
    p = base + jnp.arange(PAD_E, dtype=jnp.int32) % mod
    return jnp.concatenate([idx.astype(jnp.int32), p]).reshape(NS, CPT, CH)


@jax.jit
def kernel(nodes, senders, receivers, grid_senders, grid_receivers,
           W1, b1, W2, b2, W3, b3):
    s1 = _pad_idx(senders, 0, N)
    r1 = _pad_idx(receivers, N, N_PAD - N)
    s2 = _pad_idx(grid_senders, 0, N)
    r2 = _pad_idx(grid_receivers, N, N_PAD - N)

    h1u, h2u = _tc_mm(nodes, W1, b1, W2, b2)

    zeros1 = jnp.zeros((ROWS_PT,), jnp.float32)
    c0, c1, c2, c3 = _sc_counts(s1, r1, s2, r2, zeros1)
    cnt = jnp.stack([c0, c1, c2, c3], axis=1)

    h1s, h2s = _tc_scale(h1u, h2u, cnt)

    zeros2 = jnp.zeros((ROWS_PT, D), jnp.float32)
    agg1, agg2 = _sc_aggregate(h1s, h2s, s1, r1, s2, r2, zeros2)

    return _tc_post(agg1, agg2, h1s, h2s, cnt, W3, b3)


# speedup vs baseline: 1.1460x; 1.0001x over previous
import functools
import jax
import jax.numpy as jnp
from jax import lax
from jax.experimental import pallas as pl
from jax.experimental.pallas import tpu as pltpu
from jax.experimental.pallas import tpu_sc as plsc

N = 10000
D = 128
OUT = 128
E = 320000

NS = 16
CH = 128
G = 16
CPT = 157
NG = (CPT + G - 1) // G
LASTG = CPT - (NG - 1) * G
E_PAD = NS * CH * CPT
PAD_E = E_PAD - E
N_PAD = 10112
ROWS_PT = N_PAD // NS

BR = 1000


def _counts_body(s1, r1, s2, r2, zeros_hbm, c0, c1, c2, c3,
                 acc0, acc1, ones, idxv, zvm, semA, semB):
    c = lax.axis_index("c")
    s = lax.axis_index("s")
    for k in range(CH // 16):
        ones[pl.ds(k * 16, 16)] = jnp.ones((16,), jnp.float32)
    row = pl.ds(s * ROWS_PT, ROWS_PT)
    pltpu.sync_copy(zeros_hbm, zvm)
    pltpu.sync_copy(zvm, acc0.at[row])
    pltpu.sync_copy(zvm, acc1.at[row])
    plsc.subcore_barrier()

    def count_into(idx_hbm, acc, sems):
        pltpu.sync_copy(idx_hbm.at[s], idxv)

        pend = [None, None]
        for g in range(NG):
            pend[g % 2] = [
                pltpu.async_copy(ones, acc.at[idxv.at[g * G + j]],
                                 sems[g % 2], add=True)
                for j in range(G if g < NG - 1 else LASTG)
            ]
            if g > 0:
                for p in pend[(g - 1) % 2]:
                    p.wait()
        for p in pend[(NG - 1) % 2]:
            p.wait()

    def copyout(acc, out):
        pltpu.sync_copy(acc.at[row], zvm)
        pltpu.sync_copy(zvm, out.at[row])

    @pl.when(c == 0)
    def _():
        count_into(s1, acc0, (semA, semB))
        count_into(r1, acc1, (semA, semB))
        plsc.subcore_barrier()
        copyout(acc0, c0)
        copyout(acc1, c1)

    @pl.when(c == 1)
    def _():
        count_into(s2, acc0, (semA, semB))
        count_into(r2, acc1, (semA, semB))
        plsc.subcore_barrier()
        copyout(acc0, c2)
        copyout(acc1, c3)


def _sc_counts(s1, r1, s2, r2, zeros1):
    mesh = plsc.VectorSubcoreMesh(core_axis_name="c", subcore_axis_name="s")
    f = pl.kernel(
        _counts_body,
        out_type=[jax.ShapeDtypeStruct((N_PAD,), jnp.float32)] * 4,
        mesh=mesh,
        scratch_types=[
            pltpu.VMEM_SHARED((N_PAD,), jnp.float32),
            pltpu.VMEM_SHARED((N_PAD,), jnp.float32),
            pltpu.VMEM((CH,), jnp.float32),
            pltpu.VMEM((CPT, CH), jnp.int32),
            pltpu.VMEM((ROWS_PT,), jnp.float32),
            pltpu.SemaphoreType.DMA,
            pltpu.SemaphoreType.DMA,
        ],
    )
    return f(s1, r1, s2, r2, zeros1)


def _agg_body(h1, h2, s1, r1, s2, r2, zeros2, out1, out2,
              acc, sbuf0, sbuf1, rbuf0, rbuf1, rows0, rows1,
              gsem0, gsem1, ssem0, ssem1, isem0, isem1):
    c = lax.axis_index("c")
    s = lax.axis_index("s")
    row = pl.ds(s * ROWS_PT, ROWS_PT)
    pltpu.sync_copy(zeros2, acc.at[row])
    rows = (rows0, rows1)
    gsem = (gsem0, gsem1)
    ssem = (ssem0, ssem1)
    sbuf = (sbuf0, sbuf1)
    rbuf = (rbuf0, rbuf1)

    def run(h, sidx, ridx, out):
        pltpu.sync_copy(sidx.at[s, pl.ds(0, G)], sbuf[0])
        pltpu.sync_copy(ridx.at[s, pl.ds(0, G)], rbuf[0])
        plsc.subcore_barrier()
        pend_g = [None, None]
        pend_s = [None, None]
        pend_i = [None, None]
        pend_g[0] = pltpu.async_copy(h.at[sbuf[0].at[0]], rows[0], gsem[0])
        for jj in range(CPT):
            g, j = divmod(jj, G)
            ib = sbuf[g % 2], rbuf[g % 2]
            if j == 0 and g + 1 < NG:
                gl = G if g + 1 < NG - 1 else LASTG
                pend_i[0] = pltpu.async_copy(
                    sidx.at[s, pl.ds((g + 1) * G, gl)],
                    sbuf[(g + 1) % 2].at[pl.ds(0, gl)], isem0)
                pend_i[1] = pltpu.async_copy(
                    ridx.at[s, pl.ds((g + 1) * G, gl)],
                    rbuf[(g + 1) % 2].at[pl.ds(0, gl)], isem1)
            if jj >= 1:
                pend_s[(jj - 1) % 2].wait()
            if jj + 1 < CPT:
                ng, nj = divmod(jj + 1, G)
                if nj == 0:
                    pend_i[0].wait()
                    pend_i[1].wait()
                pend_g[(jj + 1) % 2] = pltpu.async_copy(
                    h.at[sbuf[ng % 2].at[nj]], rows[(jj + 1) % 2],
                    gsem[(jj + 1) % 2])
            pend_g[jj % 2].wait()
            pend_s[jj % 2] = pltpu.async_copy(
                rows[jj % 2], acc.at[ib[1].at[j]], ssem[jj % 2], add=True)
        pend_s[(CPT - 1) % 2].wait()
        plsc.subcore_barrier()
        pltpu.sync_copy(acc.at[row], out.at[row])

    @pl.when(c == 0)
    def _():
        run(h1, s1, r1, out1)

    @pl.when(c == 1)
    def _():
        run(h2, s2, r2, out2)


def _sc_aggregate(h1s, h2s, s1, r1, s2, r2, zeros2):
    mesh = plsc.VectorSubcoreMesh(core_axis_name="c", subcore_axis_name="s")
    f = pl.kernel(
        _agg_body,
        out_type=[jax.ShapeDtypeStruct((N_PAD, D), jnp.float32)] * 2,
        mesh=mesh,
        scratch_types=[
            pltpu.VMEM_SHARED((N_PAD, D), jnp.float32),
            pltpu.VMEM((G, CH), jnp.int32),
            pltpu.VMEM((G, CH), jnp.int32),
            pltpu.VMEM((G, CH), jnp.int32),
            pltpu.VMEM((G, CH), jnp.int32),
            pltpu.VMEM((CH, D), jnp.float32),
            pltpu.VMEM((CH, D), jnp.float32),
            pltpu.SemaphoreType.DMA,
            pltpu.SemaphoreType.DMA,
            pltpu.SemaphoreType.DMA,
            pltpu.SemaphoreType.DMA,
            pltpu.SemaphoreType.DMA,
            pltpu.SemaphoreType.DMA,
        ],
    )
    return f(h1s, h2s, s1, r1, s2, r2, zeros2)


def _mm_body(nodes, w1, b1, w2, b2, o1, o2):
    x = nodes[...]
    o1[...] = jnp.dot(x, w1[...], preferred_element_type=jnp.float32) + b1[...]
    o2[...] = jnp.dot(x, w2[...], preferred_element_type=jnp.float32) + b2[...]


def _tc_mm(nodes, W1, b1, W2, b2):
    grid = (N // BR,)
    rb = pl.BlockSpec((BR, D), lambda i: (i, 0))
    full = pl.BlockSpec((D, OUT), lambda i: (0, 0))
    bias = pl.BlockSpec((1, OUT), lambda i: (0, 0))
    ob = pl.BlockSpec((BR, OUT), lambda i: (i, 0))
    return pl.pallas_call(
        _mm_body,
        grid=grid,
        in_specs=[rb, full, bias, full, bias],
        out_specs=[ob, ob],
        out_shape=[jax.ShapeDtypeStruct((N, OUT), jnp.float32)] * 2,
    )(nodes, W1, b1.reshape(1, OUT), W2, b2.reshape(1, OUT))


def _scale_body(h1u, h2u, cnt, o1, o2):
    i = pl.program_id(0)
    gidx = i * BR + lax.broadcasted_iota(jnp.int32, (BR, 1), 0)
    corr = jnp.where(gidx < PAD_E, 1.0, 0.0).astype(jnp.float32)
    o1[...] = h1u[...] * lax.rsqrt(cnt[:, 0:1] - corr + 1.0)
    o2[...] = h2u[...] * lax.rsqrt(cnt[:, 2:3] - corr + 1.0)


def _tc_scale(h1u, h2u, cnt):
    grid = (N // BR,)
    hb = pl.BlockSpec((BR, OUT), lambda i: (i, 0))
    cb = pl.BlockSpec((BR, 4), lambda i: (i, 0))
    return pl.pallas_call(
        _scale_body,
        grid=grid,
        in_specs=[hb, hb, cb],
        out_specs=[hb, hb],
        out_shape=[jax.ShapeDtypeStruct((N, OUT), jnp.float32)] * 2,
    )(h1u, h2u, cnt)


def _post_body(agg1, agg2, h1s, h2s, cnt, w3, b3, out):
    r1 = lax.rsqrt(cnt[:, 1:2] + 1.0)
    r2 = lax.rsqrt(cnt[:, 3:4] + 1.0)
    a1 = (agg1[...] + h1s[...]) * r1
    a2 = (agg2[...] + h2s[...]) * r2
    y = jnp.dot(a1, w3[0:OUT, :], preferred_element_type=jnp.float32)
    y = y + jnp.dot(a2, w3[OUT:2 * OUT, :], preferred_element_type=jnp.float32)
    out[...] = jnp.maximum(y + b3[...], 0.0)


def _tc_post(agg1, agg2, h1s, h2s, cnt, W3, b3):
    grid = (N // BR,)
    ab = pl.BlockSpec((BR, D), lambda i: (i, 0))
    cb = pl.BlockSpec((BR, 4), lambda i: (i, 0))
    wb = pl.BlockSpec((2 * OUT, OUT), lambda i: (0, 0))
    bias = pl.BlockSpec((1, OUT), lambda i: (0, 0))
    ob = pl.BlockSpec((BR, OUT), lambda i: (i, 0))
    return pl.pallas_call(
        _post_body,
        grid=grid,
        in_specs=[ab, ab, ab, ab, cb, wb, bias],
        out_specs=ob,
        out_shape=jax.ShapeDtypeStruct((N, OUT), jnp.float32),
    )(agg1, agg2, h1s, h2s, cnt, W3, b3.reshape(1, OUT))


def _pad_idx(idx, base, mod):
    p = base + jnp.arange(PAD_E, dtype=jnp.int32) % mod
    return jnp.concatenate([idx.astype(jnp.int32), p]).reshape(NS, CPT, CH)


@jax.jit
def kernel(nodes, senders, receivers, grid_senders, grid_receivers,
           W1, b1, W2, b2, W3, b3):
    s1 = _pad_idx(senders, 0, N)
    r1 = _pad_idx(receivers, N, N_PAD - N)
    s2 = _pad_idx(grid_senders, 0, N)
    r2 = _pad_idx(grid_receivers, N, N_PAD - N)

    h1u, h2u = _tc_mm(nodes, W1, b1, W2, b2)

    zeros1 = jnp.zeros((ROWS_PT,), jnp.float32)
    c0, c1, c2, c3 = _sc_counts(s1, r1, s2, r2, zeros1)
    cnt = jnp.stack([c0, c1, c2, c3], axis=1)

    h1s, h2s = _tc_scale(h1u, h2u, cnt)

    zeros2 = jnp.zeros((ROWS_PT, D), jnp.float32)
    agg1, agg2 = _sc_aggregate(h1s, h2s, s1, r1, s2, r2, zeros2)

    return _tc_post(agg1, agg2, h1s, h2s, cnt, W3, b3)
